# baseline (speedup 1.0000x reference)
import math

import jax
import jax.numpy as jnp
import numpy as np
from jax.experimental import pallas as pl
from jax.experimental.pallas import tpu as pltpu

N = 8192
D = 5
L = 4
B = 1024
NB = N // B
TRASH = NB

_LOG2E = math.log2(math.e)

_PAIRS = [(i, j) for i in range(NB) for j in range(i, NB)]
NP_ = len(_PAIRS)
GRID = NP_ + 1


def _make_pair_table():
    tab = np.zeros((5, GRID), dtype=np.int32)
    for p in range(GRID):
        ci, cj = _PAIRS[p] if p < NP_ else (0, 0)
        tab[0, p], tab[1, p] = ci, cj
        if p == 0:
            tab[2, p], tab[3, p], tab[4, p] = TRASH, 0, TRASH
        else:
            i, j = _PAIRS[p - 1]
            tab[2, p] = i
            tab[3, p] = j
            tab[4, p] = j if j != i else TRASH
    return tab


_PAIR_TABLE = _make_pair_table()


def _body(tab_ref, ref3_ref, refT3_ref, u_ref, out_ref,
          us_ref, usT_ref, exrow_ref, acc_ref, ebuf_ref):
    p = pl.program_id(0)

    @pl.when(p == 0)
    def _init():
        rT = jnp.concatenate([refT3_ref[k] for k in range(NB)], axis=1)
        sq = jnp.sum(rT * rT, axis=0, keepdims=True)
        ex = jnp.exp2((-_LOG2E) * sq)
        exrow_ref[...] = ex
        exc = jnp.transpose(ex, (1, 0))
        us = u_ref[...] * exc
        usb = us.astype(jnp.bfloat16)
        us_ref[...] = usb.reshape(NB, B, L)
        usT = jnp.transpose(usb, (1, 0))
        for k in range(NB):
            usT_ref[k] = usT[:, k * B:(k + 1) * B]
        acc_ref[...] = jnp.zeros((NB + 1, B, L), jnp.float32)

    ci = tab_ref[0, p]
    cj = tab_ref[1, p]
    mi = tab_ref[2, p]
    mj = tab_ref[3, p]
    mjc = tab_ref[4, p]

    a = ref3_ref[ci]
    a2 = a * (2.0 * _LOG2E)
    bt = refT3_ref[cj]
    acc = a2[:, 0][:, None] * bt[0, :][None, :]
    for d in range(1, D):
        acc = acc + a2[:, d][:, None] * bt[d, :][None, :]
    ebuf_ref[p % 2] = jnp.exp2(acc).astype(jnp.bfloat16)

    e = ebuf_ref[(p + 1) % 2]
    usj = us_ref[mj]
    o = jnp.dot(e, usj, preferred_element_type=jnp.float32)
    acc_ref[mi] += o
    usti = usT_ref[mi * (mi != TRASH)]
    c = jnp.dot(usti, e, preferred_element_type=jnp.float32)
    acc_ref[mjc] += jnp.transpose(c, (1, 0))

    @pl.when(p == GRID - 1)
    def _finalize():
        v = acc_ref[0:NB].reshape(N, L)
        exc = jnp.transpose(exrow_ref[...], (1, 0))
        out_ref[...] = v * exc


@jax.jit
def kernel(U, ref):
    n, d = ref.shape
    l = U.shape[1]
    ref3 = ref.reshape(NB, B, d)
    refT3 = jnp.transpose(ref3, (0, 2, 1))

    out = pl.pallas_call(
        _body,
        grid=(GRID,),
        in_specs=[
            pl.BlockSpec(memory_space=pltpu.SMEM),
            pl.BlockSpec((NB, B, d), lambda p: (0, 0, 0)),
            pl.BlockSpec((NB, d, B), lambda p: (0, 0, 0)),
            pl.BlockSpec((n, l), lambda p: (0, 0)),
        ],
        out_specs=pl.BlockSpec((n, l), lambda p: (0, 0)),
        out_shape=jax.ShapeDtypeStruct((n, l), jnp.float32),
        scratch_shapes=[
            pltpu.VMEM((NB, B, l), jnp.bfloat16),
            pltpu.VMEM((NB, l, B), jnp.bfloat16),
            pltpu.VMEM((1, n), jnp.float32),
            pltpu.VMEM((NB + 1, B, l), jnp.float32),
            pltpu.VMEM((2, B, B), jnp.bfloat16),
        ],
    )(jnp.asarray(_PAIR_TABLE), ref3, refT3, U)
    return out


# speedup vs baseline: 1.1418x; 1.1418x over previous
import math

import jax
import jax.numpy as jnp
import numpy as np
from jax.experimental import pallas as pl
from jax.experimental.pallas import tpu as pltpu

N = 8192
D = 5
L = 4
B = 1024
NB = N // B
TRASH = NB

_LOG2E = math.log2(math.e)

_PAIRS = [(i, j) for i in range(NB) for j in range(i, NB)]
NP_ = len(_PAIRS)
NQ = NP_ + 2
GRID = NQ // 2


def _make_pair_table():
    tab = np.zeros((6, NQ), dtype=np.int32)
    for q in range(NQ):
        ci, cj = _PAIRS[q] if q < NP_ else (0, 0)
        tab[0, q], tab[1, q] = ci, cj
        if q < NP_:
            i, j = _PAIRS[q]
            tab[2, q] = i
            tab[3, q] = j
            tab[4, q] = j if j != i else TRASH
            tab[5, q] = i
        else:
            tab[2, q], tab[3, q], tab[4, q], tab[5, q] = TRASH, 0, TRASH, 0
    return tab


_PAIR_TABLE = _make_pair_table()


def _body(tab_ref, ref3_ref, refT3_ref, u_ref, out_ref,
          us_ref, usT_ref, exrow_ref, acc_ref, ebufA_ref, ebufB_ref):
    s = pl.program_id(0)

    @pl.when(s == 0)
    def _init():
        rT = jnp.concatenate([refT3_ref[k] for k in range(NB)], axis=1)
        sq = jnp.sum(rT * rT, axis=0, keepdims=True)
        ex = jnp.exp2((-_LOG2E) * sq)
        exrow_ref[...] = ex
        exc = jnp.transpose(ex, (1, 0))
        us = u_ref[...] * exc
        usb = us.astype(jnp.bfloat16)
        us_ref[...] = usb.reshape(NB, B, L)
        usT = jnp.transpose(usb, (1, 0))
        for k in range(NB):
            usT_ref[k] = usT[:, k * B:(k + 1) * B]
        acc_ref[...] = jnp.zeros((NB + 1, B, L), jnp.float32)

    def compute(q, ebuf_ref):
        ci = tab_ref[0, q]
        cj = tab_ref[1, q]
        a2 = ref3_ref[ci] * (2.0 * _LOG2E)
        bt = refT3_ref[cj]
        acc = a2[:, 0][:, None] * bt[0, :][None, :]
        for d in range(1, D):
            acc = acc + a2[:, d][:, None] * bt[d, :][None, :]
        ebuf_ref[...] = jnp.exp2(acc).astype(jnp.bfloat16)

    def matmul(q, ebuf_ref):
        mi = tab_ref[2, q]
        mj = tab_ref[3, q]
        mjc = tab_ref[4, q]
        mis = tab_ref[5, q]
        e = ebuf_ref[...]
        o = jnp.dot(e, us_ref[mj], preferred_element_type=jnp.float32)
        acc_ref[mi] += o
        c = jnp.dot(usT_ref[mis], e, preferred_element_type=jnp.float32)
        acc_ref[mjc] += jnp.transpose(c, (1, 0))

    qprev = jnp.where(s == 0, NQ - 1, 2 * s - 1)
    compute(2 * s, ebufA_ref)
    matmul(qprev, ebufB_ref)
    compute(2 * s + 1, ebufB_ref)
    matmul(2 * s, ebufA_ref)

    @pl.when(s == GRID - 1)
    def _finalize():
        v = acc_ref[0:NB].reshape(N, L)
        exc = jnp.transpose(exrow_ref[...], (1, 0))
        out_ref[...] = v * exc


@jax.jit
def kernel(U, ref):
    n, d = ref.shape
    l = U.shape[1]
    ref3 = ref.reshape(NB, B, d)
    refT3 = jnp.transpose(ref3, (0, 2, 1))

    out = pl.pallas_call(
        _body,
        grid=(GRID,),
        in_specs=[
            pl.BlockSpec(memory_space=pltpu.SMEM),
            pl.BlockSpec((NB, B, d), lambda p: (0, 0, 0)),
            pl.BlockSpec((NB, d, B), lambda p: (0, 0, 0)),
            pl.BlockSpec((n, l), lambda p: (0, 0)),
        ],
        out_specs=pl.BlockSpec((n, l), lambda p: (0, 0)),
        out_shape=jax.ShapeDtypeStruct((n, l), jnp.float32),
        scratch_shapes=[
            pltpu.VMEM((NB, B, l), jnp.bfloat16),
            pltpu.VMEM((NB, l, B), jnp.bfloat16),
            pltpu.VMEM((1, n), jnp.float32),
            pltpu.VMEM((NB + 1, B, l), jnp.float32),
            pltpu.VMEM((B, B), jnp.bfloat16),
            pltpu.VMEM((B, B), jnp.bfloat16),
        ],
    )(jnp.asarray(_PAIR_TABLE), ref3, refT3, U)
    return out


# speedup vs baseline: 1.2770x; 1.1184x over previous
import math

import jax
import jax.numpy as jnp
from jax.experimental import pallas as pl
from jax.experimental.pallas import tpu as pltpu

N = 8192
D = 5
L = 4
B = 1024
NB = N // B

_LOG2E = math.log2(math.e)


def _body(ref3_ref, refT3_ref, u_ref, out_ref,
          abig_ref, bbig_ref, us_ref, exc_ref):
    s = pl.program_id(0)

    @pl.when(s == 0)
    def _init():
        for k in range(NB):
            a = ref3_ref[k] * (2.0 * _LOG2E)
            ah = a.astype(jnp.bfloat16)
            al = (a - ah.astype(jnp.float32)).astype(jnp.bfloat16)
            zpad = jnp.zeros((B, 1), jnp.bfloat16)
            abig_ref[k] = jnp.concatenate([ah, ah, al, zpad], axis=1)
            b = refT3_ref[k]
            bh = b.astype(jnp.bfloat16)
            bl = (b - bh.astype(jnp.float32)).astype(jnp.bfloat16)
            zpad2 = jnp.zeros((1, B), jnp.bfloat16)
            bbig_ref[k] = jnp.concatenate([bh, bl, bh, zpad2], axis=0)
        rT = jnp.concatenate([refT3_ref[k] for k in range(NB)], axis=1)
        sq = jnp.sum(rT * rT, axis=0, keepdims=True)
        ex = jnp.exp2((-_LOG2E) * sq)
        exc = jnp.transpose(ex, (1, 0))
        exc_ref[...] = exc.reshape(NB, B, 1)
        us = (u_ref[...] * exc).astype(jnp.bfloat16)
        us_ref[...] = us.reshape(NB, B, L)

    asel = abig_ref[s]
    o = None
    for c in range(NB):
        g = jnp.dot(asel, bbig_ref[c],
                    preferred_element_type=jnp.float32)
        e = jnp.exp2(g).astype(jnp.bfloat16)
        oc = jnp.dot(e, us_ref[c],
                     preferred_element_type=jnp.float32)
        o = oc if o is None else o + oc
    out_ref[...] = o * exc_ref[s]


@jax.jit
def kernel(U, ref):
    n, d = ref.shape
    l = U.shape[1]
    ref3 = ref.reshape(NB, B, d)
    refT3 = jnp.transpose(ref3, (0, 2, 1))

    out = pl.pallas_call(
        _body,
        grid=(NB,),
        in_specs=[
            pl.BlockSpec((NB, B, d), lambda s: (0, 0, 0)),
            pl.BlockSpec((NB, d, B), lambda s: (0, 0, 0)),
            pl.BlockSpec((n, l), lambda s: (0, 0)),
        ],
        out_specs=pl.BlockSpec((B, l), lambda s: (s, 0)),
        out_shape=jax.ShapeDtypeStruct((n, l), jnp.float32),
        scratch_shapes=[
            pltpu.VMEM((NB, B, 16), jnp.bfloat16),
            pltpu.VMEM((NB, 16, B), jnp.bfloat16),
            pltpu.VMEM((NB, B, l), jnp.bfloat16),
            pltpu.VMEM((NB, B, 1), jnp.float32),
        ],
    )(ref3, refT3, U)
    return out
